# baseline (speedup 1.0000x reference)
import jax
import jax.numpy as jnp
from jax.experimental import pallas as pl

_K1 = 928981903
_K2 = 3453687069
_K3 = (_K1 ^ _K2 ^ 0x1BD11BDA) & 0xFFFFFFFF

_ROTS = ((13, 15, 26, 6), (17, 29, 16, 24))
_INJECT = ((1, 2, 1), (2, 0, 2), (0, 1, 3), (1, 2, 4), (2, 0, 5))


def _rotl(x, r):
    return jax.lax.shift_left(x, jnp.uint32(r)) | jax.lax.shift_right_logical(
        x, jnp.uint32(32 - r))


def _threefry_bits(l):
    ks = (jnp.uint32(_K1), jnp.uint32(_K2), jnp.uint32(_K3))
    x0 = jnp.full(l.shape, ks[0], dtype=jnp.uint32)
    x1 = l + ks[1]
    for g, (a, b, c) in enumerate(_INJECT):
        for r in _ROTS[g % 2]:
            x0 = x0 + x1
            x1 = _rotl(x1, r)
            x1 = x1 ^ x0
        x0 = x0 + ks[a]
        x1 = x1 + ks[b] + jnp.uint32(c)
    return x0 ^ x1


def _gumbel_argmax_kernel(x_ref, o_ref):
    rows, cols = x_ref.shape
    pid = pl.program_id(0)
    col = jax.lax.broadcasted_iota(jnp.int32, (rows, cols), 1)
    row = jax.lax.broadcasted_iota(jnp.int32, (rows, cols), 0) + pid * rows
    l = (row * cols + col).astype(jnp.uint32)
    bits = _threefry_bits(l)
    fb = jax.lax.shift_right_logical(bits, jnp.uint32(9)) | jnp.uint32(0x3F800000)
    u = jax.lax.bitcast_convert_type(fb, jnp.float32) - jnp.float32(1.0)
    eps = jnp.float32(1e-10)
    g = -jnp.log(-jnp.log(u + eps) + eps)
    z = x_ref[...] + g
    m = jnp.max(z, axis=1, keepdims=True)
    big = jnp.int32(2**31 - 1)
    idx = jnp.min(jnp.where(z == m, col, big), axis=1, keepdims=True)
    o_ref[...] = (col == idx).astype(jnp.float32)


@jax.jit
def kernel(logits):
    n_rows, n_cols = logits.shape
    block_rows = 8
    return pl.pallas_call(
        _gumbel_argmax_kernel,
        out_shape=jax.ShapeDtypeStruct((n_rows, n_cols), jnp.float32),
        grid=(n_rows // block_rows,),
        in_specs=[pl.BlockSpec((block_rows, n_cols), lambda i: (i, 0))],
        out_specs=pl.BlockSpec((block_rows, n_cols), lambda i: (i, 0)),
    )(logits)
